# Initial kernel scaffold; baseline (speedup 1.0000x reference)
import jax
import jax.numpy as jnp
from jax.experimental import pallas as pl


def kernel(x, edge_index, W1, b1, Wm, bm, W2, b2):
    raise NotImplementedError("write your pallas kernel here")


# speedup vs baseline: 7.7138x; 7.7138x over previous
import functools

import jax
import jax.numpy as jnp
from jax import lax
from jax.experimental import pallas as pl
from jax.experimental.pallas import tpu as pltpu
from jax.experimental.pallas import tpu_sc as plsc

N = 10000
E = 320000
D = 128

NC = 2
NS = 16
NW = NC * NS
EPT = E // NW
K = 100
NCH = EPT // K
NP = 10240
RPT = NP // NS
ZR = 16


def _sc_agg(h, src3, dst3):
    mesh = plsc.VectorSubcoreMesh(core_axis_name="c", subcore_axis_name="s")

    @functools.partial(
        pl.kernel,
        mesh=mesh,
        out_type=jax.ShapeDtypeStruct((NC, NP, D), jnp.float32),
        scratch_types=[
            pltpu.VMEM((NCH, K), jnp.int32),
            pltpu.VMEM((NCH, K), jnp.int32),
            pltpu.VMEM((K, D), jnp.float32),
            pltpu.VMEM((ZR, D), jnp.float32),
            pltpu.VMEM_SHARED((NP, D), jnp.float32),
            pltpu.SemaphoreType.DMA,
        ],
    )
    def agg_kernel(h_hbm, src_hbm, dst_hbm, out_hbm,
                   src_v, dst_v, rows_v, zero_v, acc_sh, sem):
        c = lax.axis_index("c")
        s = lax.axis_index("s")
        wid = s * NC + c

        def zrow(i, _):
            for j in range(D // 16):
                zero_v[i, pl.ds(j * 16, 16)] = jnp.zeros((16,), jnp.float32)
            return 0
        lax.fori_loop(0, ZR, zrow, 0)

        def zcp(k, _):
            pltpu.sync_copy(zero_v, acc_sh.at[pl.ds(s * RPT + k * ZR, ZR)])
            return 0
        lax.fori_loop(0, RPT // ZR, zcp, 0)

        pltpu.sync_copy(src_hbm.at[wid], src_v)
        pltpu.sync_copy(dst_hbm.at[wid], dst_v)
        plsc.subcore_barrier()

        def body(j, _):
            pltpu.async_copy(h_hbm.at[src_v.at[j]], rows_v, sem).wait()
            pltpu.sync_copy(rows_v, acc_sh.at[dst_v.at[j]], add=True)
            return 0
        lax.fori_loop(0, NCH, body, 0)
        plsc.subcore_barrier()

        pltpu.sync_copy(acc_sh.at[pl.ds(s * RPT, RPT)],
                        out_hbm.at[c, pl.ds(s * RPT, RPT)])

    return agg_kernel(h, src3, dst3)


BN = 2000


def _tc_layer(h, a0, a1, W, b, relu):
    def body(h_ref, a0_ref, a1_ref, w_ref, b_ref, o_ref):
        acc = h_ref[...] + a0_ref[...] + a1_ref[...]
        r = lax.dot_general(acc, w_ref[...], (((1,), (1,)), ((), ())),
                            preferred_element_type=jnp.float32)
        r = r + b_ref[...]
        o_ref[...] = jnp.maximum(r, 0.0) if relu else r

    row_spec = pl.BlockSpec((BN, D), lambda i: (i, 0))
    return pl.pallas_call(
        body,
        grid=(N // BN,),
        in_specs=[row_spec, row_spec, row_spec,
                  pl.BlockSpec((D, D), lambda i: (0, 0)),
                  pl.BlockSpec((1, D), lambda i: (0, 0))],
        out_specs=row_spec,
        out_shape=jax.ShapeDtypeStruct((N, D), jnp.float32),
    )(h, a0, a1, W, b.reshape(1, D))


def kernel(x, edge_index, W1, b1, Wm, bm, W2, b2):
    src3 = edge_index[0].reshape(NW, NCH, K)
    dst3 = edge_index[1].reshape(NW, NCH, K)
    h = x
    for W, b, relu in ((W1, b1, True), (Wm, bm, True), (W2, b2, False)):
        agg = _sc_agg(h, src3, dst3)
        h = _tc_layer(h, agg[0], agg[1], W, b, relu)
    return h


# speedup vs baseline: 11.0383x; 1.4310x over previous
import functools

import jax
import jax.numpy as jnp
from jax import lax
from jax.experimental import pallas as pl
from jax.experimental.pallas import tpu as pltpu
from jax.experimental.pallas import tpu_sc as plsc

N = 10000
E = 320000
D = 128

NC = 2
NS = 16
NW = NC * NS
EPT = E // NW
K = 125
NCH = EPT // K
NP = 10240
RPT = NP // NS


def _sc_agg(h, idx4):
    mesh = plsc.VectorSubcoreMesh(core_axis_name="c", subcore_axis_name="s")

    @functools.partial(
        pl.kernel,
        mesh=mesh,
        out_type=jax.ShapeDtypeStruct((NC, NP, D), jnp.float32),
        scratch_types=[
            pltpu.VMEM((2, 2, K), jnp.int32),
            pltpu.VMEM((2, K, D), jnp.float32),
            pltpu.VMEM_SHARED((NP, D), jnp.float32),
            pltpu.SemaphoreType.DMA,
            pltpu.SemaphoreType.DMA,
        ],
    )
    def agg_kernel(h_hbm, idx_hbm, out_hbm, idx_v, rows_v, acc_sh,
                   sem_g, sem_i):
        c = lax.axis_index("c")
        s = lax.axis_index("s")
        wid = s * NC + c

        def zrow(i, _):
            for j in range(D // 16):
                rows_v[0, i, pl.ds(j * 16, 16)] = jnp.zeros((16,), jnp.float32)
            return 0
        lax.fori_loop(0, K, zrow, 0)

        ZCH = (K // 8) * 8

        def zcp(k, _):
            pltpu.sync_copy(rows_v.at[0, pl.ds(0, ZCH)],
                            acc_sh.at[pl.ds(s * RPT + k * ZCH, ZCH)])
            return 0
        lax.fori_loop(0, RPT // ZCH, zcp, 0)
        if RPT % ZCH:
            pltpu.sync_copy(rows_v.at[0, pl.ds(0, RPT % ZCH)],
                            acc_sh.at[pl.ds(s * RPT + (RPT // ZCH) * ZCH,
                                            RPT % ZCH)])
        plsc.subcore_barrier()

        pltpu.sync_copy(idx_hbm.at[wid, 0], idx_v.at[0])
        pltpu.async_copy(h_hbm.at[idx_v.at[0, 0]], rows_v.at[0], sem_g)

        def body(j, _):
            buf = lax.rem(j, 2)

            @pl.when(j + 1 < NCH)
            def _():
                pltpu.async_copy(idx_hbm.at[wid, j + 1],
                                 idx_v.at[1 - buf], sem_i)

            pltpu.make_async_copy(h_hbm.at[idx_v.at[buf, 0]],
                                  rows_v.at[buf], sem_g).wait()

            @pl.when(j + 1 < NCH)
            def _():
                pltpu.make_async_copy(idx_hbm.at[wid, j + 1],
                                      idx_v.at[1 - buf], sem_i).wait()
                pltpu.async_copy(h_hbm.at[idx_v.at[1 - buf, 0]],
                                 rows_v.at[1 - buf], sem_g)

            pltpu.sync_copy(rows_v.at[buf], acc_sh.at[idx_v.at[buf, 1]],
                            add=True)
            return 0
        lax.fori_loop(0, NCH, body, 0)
        plsc.subcore_barrier()

        pltpu.sync_copy(acc_sh.at[pl.ds(s * RPT, RPT)],
                        out_hbm.at[c, pl.ds(s * RPT, RPT)])

    return agg_kernel(h, idx4)


BN = 2000


def _tc_layer(h, a0, a1, W, b, relu):
    def body(h_ref, a0_ref, a1_ref, w_ref, b_ref, o_ref):
        acc = h_ref[...] + a0_ref[...] + a1_ref[...]
        r = lax.dot_general(acc, w_ref[...], (((1,), (1,)), ((), ())),
                            preferred_element_type=jnp.float32)
        r = r + b_ref[...]
        o_ref[...] = jnp.maximum(r, 0.0) if relu else r

    row_spec = pl.BlockSpec((BN, D), lambda i: (i, 0))
    return pl.pallas_call(
        body,
        grid=(N // BN,),
        in_specs=[row_spec, row_spec, row_spec,
                  pl.BlockSpec((D, D), lambda i: (0, 0)),
                  pl.BlockSpec((1, D), lambda i: (0, 0))],
        out_specs=row_spec,
        out_shape=jax.ShapeDtypeStruct((N, D), jnp.float32),
    )(h, a0, a1, W, b.reshape(1, D))


def kernel(x, edge_index, W1, b1, Wm, bm, W2, b2):
    idx4 = edge_index.reshape(2, NW, NCH, K).transpose(1, 2, 0, 3)
    h = x
    for W, b, relu in ((W1, b1, True), (Wm, bm, True), (W2, b2, False)):
        agg = _sc_agg(h, idx4)
        h = _tc_layer(h, agg[0], agg[1], W, b, relu)
    return h


# speedup vs baseline: 14.4963x; 1.3133x over previous
import functools

import jax
import jax.numpy as jnp
from jax import lax
from jax.experimental import pallas as pl
from jax.experimental.pallas import tpu as pltpu
from jax.experimental.pallas import tpu_sc as plsc

N = 10000
E = 320000
D = 128

NC = 2
NS = 16
NW = NC * NS
EPT = E // NW
K = 125
NCH = EPT // K
NP = 10112
RPT = NP // NS


def _sc_agg(h, idx4):
    mesh = plsc.VectorSubcoreMesh(core_axis_name="c", subcore_axis_name="s")

    @functools.partial(
        pl.kernel,
        mesh=mesh,
        out_type=jax.ShapeDtypeStruct((NC, NP, D), jnp.float32),
        scratch_types=[
            pltpu.VMEM((4, 2, K), jnp.int32),
            pltpu.VMEM((3, K, D), jnp.float32),
            pltpu.VMEM_SHARED((NP, D), jnp.float32),
            pltpu.SemaphoreType.DMA((3,)),
            pltpu.SemaphoreType.DMA((4,)),
        ],
    )
    def agg_kernel(h_hbm, idx_hbm, out_hbm, idx_v, rows_v, acc_sh,
                   sem_g, sem_i):
        c = lax.axis_index("c")
        s = lax.axis_index("s")
        wid = s * NC + c

        def zrow(i, _):
            for j in range(D // 16):
                rows_v[0, i, pl.ds(j * 16, 16)] = jnp.zeros((16,), jnp.float32)
            return 0
        lax.fori_loop(0, K, zrow, 0)

        ZCH = (K // 8) * 8

        def zcp(k, _):
            pltpu.sync_copy(rows_v.at[0, pl.ds(0, ZCH)],
                            acc_sh.at[pl.ds(s * RPT + k * ZCH, ZCH)])
            return 0
        lax.fori_loop(0, RPT // ZCH, zcp, 0)
        if RPT % ZCH:
            pltpu.sync_copy(rows_v.at[0, pl.ds(0, RPT % ZCH)],
                            acc_sh.at[pl.ds(s * RPT + (RPT // ZCH) * ZCH,
                                            RPT % ZCH)])
        plsc.subcore_barrier()

        pltpu.sync_copy(idx_hbm.at[wid, 0], idx_v.at[0])
        pltpu.sync_copy(idx_hbm.at[wid, 1], idx_v.at[1])
        pltpu.async_copy(idx_hbm.at[wid, 2], idx_v.at[2], sem_i.at[2])
        pltpu.async_copy(h_hbm.at[idx_v.at[0, 0]], rows_v.at[0],
                         sem_g.at[0])
        pltpu.async_copy(h_hbm.at[idx_v.at[1, 0]], rows_v.at[1],
                         sem_g.at[1])

        def body(j, _):
            buf = lax.rem(j, 3)
            ib = lax.rem(j, 4)
            ib2 = lax.rem(j + 2, 4)
            ib3 = lax.rem(j + 3, 4)

            @pl.when(j + 3 < NCH)
            def _():
                pltpu.async_copy(idx_hbm.at[wid, j + 3], idx_v.at[ib3],
                                 sem_i.at[ib3])

            @pl.when(j + 2 < NCH)
            def _():
                pltpu.make_async_copy(idx_hbm.at[wid, j + 2],
                                      idx_v.at[ib2], sem_i.at[ib2]).wait()
                pltpu.async_copy(h_hbm.at[idx_v.at[ib2, 0]],
                                 rows_v.at[lax.rem(j + 2, 3)],
                                 sem_g.at[lax.rem(j + 2, 3)])

            pltpu.make_async_copy(h_hbm.at[idx_v.at[ib, 0]],
                                  rows_v.at[buf], sem_g.at[buf]).wait()
            pltpu.sync_copy(rows_v.at[buf], acc_sh.at[idx_v.at[ib, 1]],
                            add=True)
            return 0
        lax.fori_loop(0, NCH, body, 0)
        plsc.subcore_barrier()

        pltpu.sync_copy(acc_sh.at[pl.ds(s * RPT, RPT)],
                        out_hbm.at[c, pl.ds(s * RPT, RPT)])

    return agg_kernel(h, idx4)


BN = 2000


def _tc_layer(h, a0, a1, W, b, relu):
    def body(h_ref, a0_ref, a1_ref, w_ref, b_ref, o_ref):
        acc = h_ref[...] + a0_ref[...] + a1_ref[...]
        r = lax.dot_general(acc, w_ref[...], (((1,), (1,)), ((), ())),
                            preferred_element_type=jnp.float32)
        r = r + b_ref[...]
        o_ref[...] = jnp.maximum(r, 0.0) if relu else r

    row_spec = pl.BlockSpec((BN, D), lambda i: (i, 0))
    return pl.pallas_call(
        body,
        grid=(N // BN,),
        in_specs=[row_spec, row_spec, row_spec,
                  pl.BlockSpec((D, D), lambda i: (0, 0)),
                  pl.BlockSpec((1, D), lambda i: (0, 0))],
        out_specs=row_spec,
        out_shape=jax.ShapeDtypeStruct((N, D), jnp.float32),
    )(h, a0, a1, W, b.reshape(1, D))


def kernel(x, edge_index, W1, b1, Wm, bm, W2, b2):
    idx4 = edge_index.reshape(2, NW, NCH, K).transpose(1, 2, 0, 3)
    h = x
    for W, b, relu in ((W1, b1, True), (Wm, bm, True), (W2, b2, False)):
        agg = _sc_agg(h, idx4)
        h = _tc_layer(h, agg[0], agg[1], W, b, relu)
    return h


# speedup vs baseline: 15.4359x; 1.0648x over previous
import functools

import jax
import jax.numpy as jnp
from jax import lax
from jax.experimental import pallas as pl
from jax.experimental.pallas import tpu as pltpu
from jax.experimental.pallas import tpu_sc as plsc

N = 10000
E = 320000
D = 128

NC = 2
NS = 16
NW = NC * NS
EPT = E // NW
K = 125
NCH = EPT // K
NP = 10112
RPT = NP // NS


def _sc_agg(h, idx4):
    mesh = plsc.VectorSubcoreMesh(core_axis_name="c", subcore_axis_name="s")

    @functools.partial(
        pl.kernel,
        mesh=mesh,
        out_type=jax.ShapeDtypeStruct((NC, NP, D), jnp.float32),
        scratch_types=[
            pltpu.VMEM((4, 2, K), jnp.int32),
            pltpu.VMEM((3, K, D), jnp.float32),
            pltpu.VMEM_SHARED((NP, D), jnp.float32),
            pltpu.SemaphoreType.DMA((3,)),
            pltpu.SemaphoreType.DMA((4,)),
        ],
    )
    def agg_kernel(h_hbm, idx_hbm, out_hbm, idx_v, rows_v, acc_sh,
                   sem_g, sem_i):
        c = lax.axis_index("c")
        s = lax.axis_index("s")
        wid = s * NC + c

        def zrow(i, _):
            for j in range(D // 16):
                rows_v[0, i, pl.ds(j * 16, 16)] = jnp.zeros((16,), jnp.float32)
            return 0
        lax.fori_loop(0, K, zrow, 0)

        ZCH = (K // 8) * 8

        def zcp(k, _):
            pltpu.sync_copy(rows_v.at[0, pl.ds(0, ZCH)],
                            acc_sh.at[pl.ds(s * RPT + k * ZCH, ZCH)])
            return 0
        lax.fori_loop(0, RPT // ZCH, zcp, 0)
        if RPT % ZCH:
            pltpu.sync_copy(rows_v.at[0, pl.ds(0, RPT % ZCH)],
                            acc_sh.at[pl.ds(s * RPT + (RPT // ZCH) * ZCH,
                                            RPT % ZCH)])
        plsc.subcore_barrier()

        pltpu.sync_copy(idx_hbm.at[wid, 0], idx_v.at[0])
        pltpu.sync_copy(idx_hbm.at[wid, 1], idx_v.at[1])
        pltpu.async_copy(idx_hbm.at[wid, 2], idx_v.at[2], sem_i.at[2])
        pltpu.async_copy(h_hbm.at[idx_v.at[0, 0]], rows_v.at[0],
                         sem_g.at[0])
        pltpu.async_copy(h_hbm.at[idx_v.at[1, 0]], rows_v.at[1],
                         sem_g.at[1])

        def body(j, _):
            buf = lax.rem(j, 3)
            ib = lax.rem(j, 4)
            ib2 = lax.rem(j + 2, 4)
            ib3 = lax.rem(j + 3, 4)

            @pl.when(j + 3 < NCH)
            def _():
                pltpu.async_copy(idx_hbm.at[wid, j + 3], idx_v.at[ib3],
                                 sem_i.at[ib3])

            @pl.when(j + 2 < NCH)
            def _():
                pltpu.make_async_copy(idx_hbm.at[wid, j + 2],
                                      idx_v.at[ib2], sem_i.at[ib2]).wait()
                pltpu.async_copy(h_hbm.at[idx_v.at[ib2, 0]],
                                 rows_v.at[lax.rem(j + 2, 3)],
                                 sem_g.at[lax.rem(j + 2, 3)])

            pltpu.make_async_copy(h_hbm.at[idx_v.at[ib, 0]],
                                  rows_v.at[buf], sem_g.at[buf]).wait()
            pltpu.sync_copy(rows_v.at[buf], acc_sh.at[idx_v.at[ib, 1]],
                            add=True)
            return 0
        lax.fori_loop(0, NCH, body, 0)
        plsc.subcore_barrier()

        pltpu.sync_copy(acc_sh.at[pl.ds(s * RPT, RPT)],
                        out_hbm.at[c, pl.ds(s * RPT, RPT)])

    return agg_kernel(h, idx4)


BN = 2000


def _tc_layer(h, agg, W, b, relu):
    def body(h_ref, a_ref, w_ref, b_ref, o_ref):
        acc = h_ref[...] + a_ref[0] + a_ref[1]
        r = lax.dot_general(acc, w_ref[...], (((1,), (1,)), ((), ())),
                            preferred_element_type=jnp.float32)
        r = r + b_ref[...]
        o_ref[...] = jnp.maximum(r, 0.0) if relu else r

    row_spec = pl.BlockSpec((BN, D), lambda i: (i, 0))
    return pl.pallas_call(
        body,
        grid=(N // BN,),
        in_specs=[row_spec,
                  pl.BlockSpec((2, BN, D), lambda i: (0, i, 0)),
                  pl.BlockSpec((D, D), lambda i: (0, 0)),
                  pl.BlockSpec((1, D), lambda i: (0, 0))],
        out_specs=row_spec,
        out_shape=jax.ShapeDtypeStruct((N, D), jnp.float32),
    )(h, agg, W, b.reshape(1, D))


def kernel(x, edge_index, W1, b1, Wm, bm, W2, b2):
    idx4 = edge_index.reshape(2, NW, NCH, K).transpose(1, 2, 0, 3)
    h = x
    for W, b, relu in ((W1, b1, True), (Wm, bm, True), (W2, b2, False)):
        agg = _sc_agg(h, idx4)
        h = _tc_layer(h, agg, W, b, relu)
    return h


# speedup vs baseline: 15.5359x; 1.0065x over previous
import functools

import jax
import jax.numpy as jnp
from jax import lax
from jax.experimental import pallas as pl
from jax.experimental.pallas import tpu as pltpu
from jax.experimental.pallas import tpu_sc as plsc

N = 10000
E = 320000
D = 128

NC = 2
NS = 16
NW = NC * NS
EPT = E // NW
K = 125
NCH = EPT // K
NP = 10112
RPT = NP // NS


def _sc_agg(h, idx4):
    mesh = plsc.VectorSubcoreMesh(core_axis_name="c", subcore_axis_name="s")

    @functools.partial(
        pl.kernel,
        mesh=mesh,
        out_type=jax.ShapeDtypeStruct((NC, NP, D), jnp.float32),
        scratch_types=[
            pltpu.VMEM((4, 2, K), jnp.int32),
            pltpu.VMEM((3, K, D), jnp.float32),
            pltpu.VMEM_SHARED((NP, D), jnp.float32),
            pltpu.SemaphoreType.DMA((3,)),
            pltpu.SemaphoreType.DMA((4,)),
        ],
    )
    def agg_kernel(h_hbm, idx_hbm, out_hbm, idx_v, rows_v, acc_sh,
                   sem_g, sem_i):
        c = lax.axis_index("c")
        s = lax.axis_index("s")
        wid = s * NC + c

        pltpu.async_copy(idx_hbm.at[wid, 0], idx_v.at[0], sem_i.at[0])
        pltpu.async_copy(idx_hbm.at[wid, 1], idx_v.at[1], sem_i.at[1])
        pltpu.async_copy(idx_hbm.at[wid, 2], idx_v.at[2], sem_i.at[2])

        def zrow(i, _):
            for j in range(D // 16):
                rows_v[0, i, pl.ds(j * 16, 16)] = jnp.zeros((16,), jnp.float32)
            return 0
        lax.fori_loop(0, K, zrow, 0)

        ZCH = (K // 8) * 8

        def zcp(k, _):
            pltpu.sync_copy(rows_v.at[0, pl.ds(0, ZCH)],
                            acc_sh.at[pl.ds(s * RPT + k * ZCH, ZCH)])
            return 0
        lax.fori_loop(0, RPT // ZCH, zcp, 0)
        if RPT % ZCH:
            pltpu.sync_copy(rows_v.at[0, pl.ds(0, RPT % ZCH)],
                            acc_sh.at[pl.ds(s * RPT + (RPT // ZCH) * ZCH,
                                            RPT % ZCH)])

        pltpu.make_async_copy(idx_hbm.at[wid, 0], idx_v.at[0],
                              sem_i.at[0]).wait()
        pltpu.async_copy(h_hbm.at[idx_v.at[0, 0]], rows_v.at[0],
                         sem_g.at[0])
        pltpu.make_async_copy(idx_hbm.at[wid, 1], idx_v.at[1],
                              sem_i.at[1]).wait()
        pltpu.async_copy(h_hbm.at[idx_v.at[1, 0]], rows_v.at[1],
                         sem_g.at[1])
        plsc.subcore_barrier()

        def body(j, _):
            buf = lax.rem(j, 3)
            ib = lax.rem(j, 4)
            ib2 = lax.rem(j + 2, 4)
            ib3 = lax.rem(j + 3, 4)

            @pl.when(j + 3 < NCH)
            def _():
                pltpu.async_copy(idx_hbm.at[wid, j + 3], idx_v.at[ib3],
                                 sem_i.at[ib3])

            @pl.when(j + 2 < NCH)
            def _():
                pltpu.make_async_copy(idx_hbm.at[wid, j + 2],
                                      idx_v.at[ib2], sem_i.at[ib2]).wait()
                pltpu.async_copy(h_hbm.at[idx_v.at[ib2, 0]],
                                 rows_v.at[lax.rem(j + 2, 3)],
                                 sem_g.at[lax.rem(j + 2, 3)])

            pltpu.make_async_copy(h_hbm.at[idx_v.at[ib, 0]],
                                  rows_v.at[buf], sem_g.at[buf]).wait()
            pltpu.sync_copy(rows_v.at[buf], acc_sh.at[idx_v.at[ib, 1]],
                            add=True)
            return 0
        lax.fori_loop(0, NCH, body, 0)
        plsc.subcore_barrier()

        pltpu.sync_copy(acc_sh.at[pl.ds(s * RPT, RPT)],
                        out_hbm.at[c, pl.ds(s * RPT, RPT)])

    return agg_kernel(h, idx4)


BN = 2000


def _tc_layer(h, agg, W, b, relu):
    def body(h_ref, a_ref, w_ref, b_ref, o_ref):
        acc = h_ref[...] + a_ref[0] + a_ref[1]
        r = lax.dot_general(acc, w_ref[...], (((1,), (1,)), ((), ())),
                            preferred_element_type=jnp.float32)
        r = r + b_ref[...]
        o_ref[...] = jnp.maximum(r, 0.0) if relu else r

    row_spec = pl.BlockSpec((BN, D), lambda i: (i, 0))
    return pl.pallas_call(
        body,
        grid=(N // BN,),
        in_specs=[row_spec,
                  pl.BlockSpec((2, BN, D), lambda i: (0, i, 0)),
                  pl.BlockSpec((D, D), lambda i: (0, 0)),
                  pl.BlockSpec((1, D), lambda i: (0, 0))],
        out_specs=row_spec,
        out_shape=jax.ShapeDtypeStruct((N, D), jnp.float32),
    )(h, agg, W, b.reshape(1, D))


def kernel(x, edge_index, W1, b1, Wm, bm, W2, b2):
    idx4 = edge_index.reshape(2, NW, NCH, K).transpose(1, 2, 0, 3)
    h = x
    for W, b, relu in ((W1, b1, True), (Wm, bm, True), (W2, b2, False)):
        agg = _sc_agg(h, idx4)
        h = _tc_layer(h, agg, W, b, relu)
    return h
